# Initial kernel scaffold; baseline (speedup 1.0000x reference)
import jax
import jax.numpy as jnp
from jax.experimental import pallas as pl


def kernel(x, edge_index, W1, att_src1, att_dst1, b1, W2, att_src2, att_dst2, b2, W3, att_src3, att_dst3, b3, W4, att_src4, att_dst4, b4):
    raise NotImplementedError("write your pallas kernel here")


# speedup vs baseline: 16.5607x; 16.5607x over previous
import functools

import jax
import jax.numpy as jnp
from jax import lax
from jax.experimental import pallas as pl
from jax.experimental.pallas import tpu as pltpu
from jax.experimental.pallas import tpu_sc as plsc

N_NODES = 10000
NP = 10240
D = 128
E_TOT = 320000 + N_NODES
NW = 32
E_LOC = 10368
EB = 1152
EP = NW * E_LOC
ROWS_PER_TILE = NP // 16
BM = 1024
DEN_EPS = 1e-16


def _proj(h, as_w, ad_w, h_ref, as_ref, ad_ref):
    h_ref[...] = h
    as_ref[...] = jnp.sum(h * as_w, axis=1)
    ad_ref[...] = jnp.sum(h * ad_w, axis=1)


def _first_dense_body(x_ref, w_ref, as_w_ref, ad_w_ref, h_ref, as_ref, ad_ref):
    h = jnp.dot(x_ref[...], w_ref[...], preferred_element_type=jnp.float32)
    _proj(h, as_w_ref[...], ad_w_ref[...], h_ref, as_ref, ad_ref)


def _mid_dense_body(n0_ref, n1_ref, d0_ref, d1_ref, b_ref, w_ref, as_w_ref,
                    ad_w_ref, h_ref, as_ref, ad_ref):
    den = d0_ref[...] + d1_ref[...] + DEN_EPS
    act = jnp.maximum((n0_ref[...] + n1_ref[...]) / den[:, None] + b_ref[...], 0.0)
    h = jnp.dot(act, w_ref[...], preferred_element_type=jnp.float32)
    _proj(h, as_w_ref[...], ad_w_ref[...], h_ref, as_ref, ad_ref)


def _final_body(n0_ref, n1_ref, d0_ref, d1_ref, b_ref, y_ref):
    den = d0_ref[...] + d1_ref[...] + DEN_EPS
    z = (n0_ref[...] + n1_ref[...]) / den[:, None] + b_ref[...]
    m = jnp.max(z, axis=1, keepdims=True)
    e = jnp.exp(z - m)
    y_ref[...] = e / jnp.sum(e, axis=1, keepdims=True)


_mat_spec = pl.BlockSpec((BM, D), lambda i: (i, 0))
_vec_spec = pl.BlockSpec((BM,), lambda i: (i,))
_row_spec = pl.BlockSpec((1, D), lambda i: (0, 0))
_w_spec = pl.BlockSpec((D, D), lambda i: (0, 0))

_dense_out = [
    jax.ShapeDtypeStruct((NP, D), jnp.float32),
    jax.ShapeDtypeStruct((NP,), jnp.float32),
    jax.ShapeDtypeStruct((NP,), jnp.float32),
]
_dense_out_spec = [_mat_spec, _vec_spec, _vec_spec]

_first_dense = pl.pallas_call(
    _first_dense_body,
    grid=(NP // BM,),
    in_specs=[_mat_spec, _w_spec, _row_spec, _row_spec],
    out_specs=_dense_out_spec,
    out_shape=_dense_out,
)

_mid_dense = pl.pallas_call(
    _mid_dense_body,
    grid=(NP // BM,),
    in_specs=[_mat_spec, _mat_spec, _vec_spec, _vec_spec, _row_spec, _w_spec,
              _row_spec, _row_spec],
    out_specs=_dense_out_spec,
    out_shape=_dense_out,
)

_final = pl.pallas_call(
    _final_body,
    grid=(NP // BM,),
    in_specs=[_mat_spec, _mat_spec, _vec_spec, _vec_spec, _row_spec],
    out_specs=_mat_spec,
    out_shape=jax.ShapeDtypeStruct((NP, D), jnp.float32),
)


_sc_mesh = plsc.VectorSubcoreMesh(core_axis_name="c", subcore_axis_name="s")


@functools.partial(
    pl.kernel,
    mesh=_sc_mesh,
    compiler_params=pltpu.CompilerParams(needs_layout_passes=False),
    out_type=[
        jax.ShapeDtypeStruct((NP, D), jnp.float32),
        jax.ShapeDtypeStruct((NP, D), jnp.float32),
        jax.ShapeDtypeStruct((NP,), jnp.float32),
        jax.ShapeDtypeStruct((NP,), jnp.float32),
    ],
    scratch_types=[
        pltpu.VMEM((EB,), jnp.int32),
        pltpu.VMEM((EB,), jnp.int32),
        pltpu.VMEM((EB // 128, 128), jnp.int32),
        pltpu.VMEM((NP,), jnp.float32),
        pltpu.VMEM((NP,), jnp.float32),
        pltpu.VMEM((EB,), jnp.float32),
        pltpu.VMEM((16, D), jnp.float32),
        pltpu.VMEM((16, D), jnp.float32),
        pltpu.VMEM_SHARED((NP, D), jnp.float32),
        pltpu.VMEM_SHARED((NP,), jnp.float32),
        pltpu.SemaphoreType.DMA,
        pltpu.SemaphoreType.DMA,
    ],
)
def _edge_kernel(src_hbm, dst_hbm, dst2_hbm, as_hbm, ad_hbm, h_hbm, z2d_hbm,
                 z1d_hbm, num0_hbm, num1_hbm, den0_hbm, den1_hbm,
                 src_v, dst_v, dst_r, as_v, ad_v, p_v, rows_v, scaled_v,
                 num_acc, den_acc, sem_g, sem_s):
    c = lax.axis_index("c")
    s = lax.axis_index("s")
    w = s * 2 + c
    base = w * E_LOC
    r0 = s * ROWS_PER_TILE

    pltpu.sync_copy(as_hbm, as_v)
    pltpu.sync_copy(ad_hbm, ad_v)
    for k in range(ROWS_PER_TILE // 64):
        pltpu.sync_copy(z2d_hbm, num_acc.at[pl.ds(r0 + k * 64, 64)])
    pltpu.sync_copy(z1d_hbm, den_acc.at[pl.ds(r0, ROWS_PER_TILE)])
    plsc.subcore_barrier()

    def chunk(i, carry):
        sv = src_v[pl.ds(i * 16, 16)]
        dv = dst_v[pl.ds(i * 16, 16)]
        a = plsc.load_gather(as_v, [sv])
        b = plsc.load_gather(ad_v, [dv])
        e = a + b
        e = jnp.where(e >= 0.0, e, 0.2 * e)
        p = jnp.exp(e)
        p_v[pl.ds(i * 16, 16)] = p
        pltpu.async_copy(h_hbm.at[sv], rows_v, sem_g).wait()
        for k in range(16):
            pk = p[k]
            for j in range(D // 16):
                scaled_v[k, pl.ds(j * 16, 16)] = rows_v[k, pl.ds(j * 16, 16)] * pk
        pltpu.async_copy(scaled_v, num_acc.at[dv], sem_s, add=True).wait()
        return carry

    def dchunk(j, carry):
        pltpu.sync_copy(p_v.at[pl.ds(j * 128, 128)], den_acc.at[dst_r.at[j]],
                        add=True)
        return carry

    def block(blk, carry):
        off = base + blk * EB
        pltpu.sync_copy(src_hbm.at[pl.ds(off, EB)], src_v)
        pltpu.sync_copy(dst_hbm.at[pl.ds(off, EB)], dst_v)
        pltpu.sync_copy(dst2_hbm.at[w, blk], dst_r)
        lax.fori_loop(0, EB // 16, chunk, 0)
        lax.fori_loop(0, EB // 128, dchunk, 0)
        return carry

    lax.fori_loop(0, E_LOC // EB, block, 0)
    plsc.subcore_barrier()

    @pl.when(c == 0)
    def _():
        pltpu.sync_copy(num_acc.at[pl.ds(r0, ROWS_PER_TILE)],
                        num0_hbm.at[pl.ds(r0, ROWS_PER_TILE)])
        pltpu.sync_copy(den_acc.at[pl.ds(r0, ROWS_PER_TILE)],
                        den0_hbm.at[pl.ds(r0, ROWS_PER_TILE)])

    @pl.when(c == 1)
    def _():
        pltpu.sync_copy(num_acc.at[pl.ds(r0, ROWS_PER_TILE)],
                        num1_hbm.at[pl.ds(r0, ROWS_PER_TILE)])
        pltpu.sync_copy(den_acc.at[pl.ds(r0, ROWS_PER_TILE)],
                        den1_hbm.at[pl.ds(r0, ROWS_PER_TILE)])


def kernel(x, edge_index, W1, att_src1, att_dst1, b1, W2, att_src2, att_dst2,
           b2, W3, att_src3, att_dst3, b3, W4, att_src4, att_dst4, b4):
    n = x.shape[0]
    loop = jnp.arange(n, dtype=jnp.int32)
    pad = EP - E_TOT
    src = jnp.concatenate([edge_index[0], loop, jnp.zeros((pad,), jnp.int32)])
    dst = jnp.concatenate([edge_index[1], loop,
                           jnp.full((pad,), NP - 1, jnp.int32)])
    dst2 = dst.reshape(NW, E_LOC // EB, EB // 128, 128)
    x_pad = jnp.concatenate([x, jnp.zeros((NP - n, D), jnp.float32)], axis=0)
    z2d = jnp.zeros((64, D), jnp.float32)
    z1d = jnp.zeros((ROWS_PER_TILE,), jnp.float32)

    def edge(a_s, a_d, h):
        return _edge_kernel(src, dst, dst2, a_s, a_d, h, z2d, z1d)

    h, a_s, a_d = _first_dense(x_pad, W1, att_src1.reshape(1, D),
                               att_dst1.reshape(1, D))
    n0, n1, d0, d1 = edge(a_s, a_d, h)
    h, a_s, a_d = _mid_dense(n0, n1, d0, d1, b1.reshape(1, D), W2,
                             att_src2.reshape(1, D), att_dst2.reshape(1, D))
    n0, n1, d0, d1 = edge(a_s, a_d, h)
    h, a_s, a_d = _mid_dense(n0, n1, d0, d1, b2.reshape(1, D), W3,
                             att_src3.reshape(1, D), att_dst3.reshape(1, D))
    n0, n1, d0, d1 = edge(a_s, a_d, h)
    h, a_s, a_d = _mid_dense(n0, n1, d0, d1, b3.reshape(1, D), W4,
                             att_src4.reshape(1, D), att_dst4.reshape(1, D))
    n0, n1, d0, d1 = edge(a_s, a_d, h)
    return _final(n0, n1, d0, d1, b4.reshape(1, D))[:n]


# speedup vs baseline: 27.9155x; 1.6856x over previous
import functools

import jax
import jax.numpy as jnp
from jax import lax
from jax.experimental import pallas as pl
from jax.experimental.pallas import tpu as pltpu
from jax.experimental.pallas import tpu_sc as plsc

N_NODES = 10000
NP = 10240
D = 128
E_TOT = 320000 + N_NODES
NW = 32
E_LOC = 10368
EB = 1152
EP = NW * E_LOC
ROWS_PER_TILE = NP // 16
BM = 1024
DEN_EPS = 1e-16


def _proj(h, as_w, ad_w, h_ref, as_ref, ad_ref):
    h_ref[...] = h
    as_ref[...] = jnp.sum(h * as_w, axis=1)
    ad_ref[...] = jnp.sum(h * ad_w, axis=1)


def _first_dense_body(x_ref, w_ref, as_w_ref, ad_w_ref, h_ref, as_ref, ad_ref):
    h = jnp.dot(x_ref[...], w_ref[...], preferred_element_type=jnp.float32)
    _proj(h, as_w_ref[...], ad_w_ref[...], h_ref, as_ref, ad_ref)


def _mid_dense_body(n0_ref, n1_ref, d0_ref, d1_ref, b_ref, w_ref, as_w_ref,
                    ad_w_ref, h_ref, as_ref, ad_ref):
    den = d0_ref[...] + d1_ref[...] + DEN_EPS
    act = jnp.maximum((n0_ref[...] + n1_ref[...]) / den[:, None] + b_ref[...], 0.0)
    h = jnp.dot(act, w_ref[...], preferred_element_type=jnp.float32)
    _proj(h, as_w_ref[...], ad_w_ref[...], h_ref, as_ref, ad_ref)


def _final_body(n0_ref, n1_ref, d0_ref, d1_ref, b_ref, y_ref):
    den = d0_ref[...] + d1_ref[...] + DEN_EPS
    z = (n0_ref[...] + n1_ref[...]) / den[:, None] + b_ref[...]
    m = jnp.max(z, axis=1, keepdims=True)
    e = jnp.exp(z - m)
    y_ref[...] = e / jnp.sum(e, axis=1, keepdims=True)


_mat_spec = pl.BlockSpec((BM, D), lambda i: (i, 0))
_vec_spec = pl.BlockSpec((BM,), lambda i: (i,))
_row_spec = pl.BlockSpec((1, D), lambda i: (0, 0))
_w_spec = pl.BlockSpec((D, D), lambda i: (0, 0))

_dense_out = [
    jax.ShapeDtypeStruct((NP, D), jnp.float32),
    jax.ShapeDtypeStruct((NP,), jnp.float32),
    jax.ShapeDtypeStruct((NP,), jnp.float32),
]
_dense_out_spec = [_mat_spec, _vec_spec, _vec_spec]

_first_dense = pl.pallas_call(
    _first_dense_body,
    grid=(NP // BM,),
    in_specs=[_mat_spec, _w_spec, _row_spec, _row_spec],
    out_specs=_dense_out_spec,
    out_shape=_dense_out,
)

_mid_dense = pl.pallas_call(
    _mid_dense_body,
    grid=(NP // BM,),
    in_specs=[_mat_spec, _mat_spec, _vec_spec, _vec_spec, _row_spec, _w_spec,
              _row_spec, _row_spec],
    out_specs=_dense_out_spec,
    out_shape=_dense_out,
)

_final = pl.pallas_call(
    _final_body,
    grid=(NP // BM,),
    in_specs=[_mat_spec, _mat_spec, _vec_spec, _vec_spec, _row_spec],
    out_specs=_mat_spec,
    out_shape=jax.ShapeDtypeStruct((NP, D), jnp.float32),
)


_sc_mesh = plsc.VectorSubcoreMesh(core_axis_name="c", subcore_axis_name="s")


@functools.partial(
    pl.kernel,
    mesh=_sc_mesh,
    compiler_params=pltpu.CompilerParams(needs_layout_passes=False),
    out_type=[
        jax.ShapeDtypeStruct((NP, D), jnp.float32),
        jax.ShapeDtypeStruct((NP, D), jnp.float32),
        jax.ShapeDtypeStruct((NP,), jnp.float32),
        jax.ShapeDtypeStruct((NP,), jnp.float32),
    ],
    scratch_types=[
        pltpu.VMEM((EB,), jnp.int32),
        pltpu.VMEM((EB,), jnp.int32),
        pltpu.VMEM((EB // 128, 128), jnp.int32),
        pltpu.VMEM((NP,), jnp.float32),
        pltpu.VMEM((NP,), jnp.float32),
        pltpu.VMEM((EB,), jnp.float32),
        pltpu.VMEM((16, D), jnp.float32),
        pltpu.VMEM((16, D), jnp.float32),
        pltpu.VMEM((16, D), jnp.float32),
        pltpu.VMEM((16, D), jnp.float32),
        pltpu.VMEM_SHARED((NP, D), jnp.float32),
        pltpu.VMEM_SHARED((NP,), jnp.float32),
        pltpu.SemaphoreType.DMA,
        pltpu.SemaphoreType.DMA,
    ],
)
def _edge_kernel(src_hbm, dst_hbm, dst2_hbm, as_hbm, ad_hbm, h_hbm, z2d_hbm,
                 z1d_hbm, num0_hbm, num1_hbm, den0_hbm, den1_hbm,
                 src_v, dst_v, dst_r, as_v, ad_v, p_v, rows_a, rows_b,
                 scaled_a, scaled_b, num_acc, den_acc, sem_a, sem_b):
    c = lax.axis_index("c")
    s = lax.axis_index("s")
    w = s * 2 + c
    base = w * E_LOC
    r0 = s * ROWS_PER_TILE

    pltpu.sync_copy(as_hbm, as_v)
    pltpu.sync_copy(ad_hbm, ad_v)
    for k in range(ROWS_PER_TILE // 64):
        pltpu.sync_copy(z2d_hbm, num_acc.at[pl.ds(r0 + k * 64, 64)])
    pltpu.sync_copy(z1d_hbm, den_acc.at[pl.ds(r0, ROWS_PER_TILE)])
    plsc.subcore_barrier()

    def issue(i, rows, sem):
        sv = src_v[pl.ds(i * 16, 16)]
        dv = dst_v[pl.ds(i * 16, 16)]
        a = plsc.load_gather(as_v, [sv])
        b = plsc.load_gather(ad_v, [dv])
        e = a + b
        e = jnp.where(e >= 0.0, e, 0.2 * e)
        p = jnp.exp(e)
        p_v[pl.ds(i * 16, 16)] = p
        pltpu.async_copy(h_hbm.at[sv], rows, sem)
        return dv, p

    def process(dv, p, rows, scaled, sem):
        pltpu.make_async_copy(h_hbm.at[dv], rows, sem).wait()
        for k in range(16):
            pk = p[k]
            for j in range(D // 16):
                scaled[k, pl.ds(j * 16, 16)] = rows[k, pl.ds(j * 16, 16)] * pk
        pltpu.async_copy(scaled, num_acc.at[dv], sem, add=True).wait()

    def dchunk(j, carry):
        pltpu.sync_copy(p_v.at[pl.ds(j * 128, 128)], den_acc.at[dst_r.at[j]],
                        add=True)
        return carry

    nchunk = EB // 16

    def block(blk, carry):
        off = base + blk * EB
        pltpu.sync_copy(src_hbm.at[pl.ds(off, EB)], src_v)
        pltpu.sync_copy(dst_hbm.at[pl.ds(off, EB)], dst_v)
        pltpu.sync_copy(dst2_hbm.at[w, blk], dst_r)
        dva, pa = issue(0, rows_a, sem_a)
        dvb, pb = issue(1, rows_b, sem_b)

        def pair(t, c):
            dva, pa, dvb, pb = c
            process(dva, pa, rows_a, scaled_a, sem_a)
            dva2, pa2 = issue(2 * t + 2, rows_a, sem_a)
            process(dvb, pb, rows_b, scaled_b, sem_b)
            dvb2, pb2 = issue(2 * t + 3, rows_b, sem_b)
            return dva2, pa2, dvb2, pb2

        dva, pa, dvb, pb = lax.fori_loop(0, nchunk // 2 - 1, pair,
                                         (dva, pa, dvb, pb))
        process(dva, pa, rows_a, scaled_a, sem_a)
        process(dvb, pb, rows_b, scaled_b, sem_b)
        lax.fori_loop(0, EB // 128, dchunk, 0)
        return carry

    lax.fori_loop(0, E_LOC // EB, block, 0)
    plsc.subcore_barrier()

    @pl.when(c == 0)
    def _():
        pltpu.sync_copy(num_acc.at[pl.ds(r0, ROWS_PER_TILE)],
                        num0_hbm.at[pl.ds(r0, ROWS_PER_TILE)])
        pltpu.sync_copy(den_acc.at[pl.ds(r0, ROWS_PER_TILE)],
                        den0_hbm.at[pl.ds(r0, ROWS_PER_TILE)])

    @pl.when(c == 1)
    def _():
        pltpu.sync_copy(num_acc.at[pl.ds(r0, ROWS_PER_TILE)],
                        num1_hbm.at[pl.ds(r0, ROWS_PER_TILE)])
        pltpu.sync_copy(den_acc.at[pl.ds(r0, ROWS_PER_TILE)],
                        den1_hbm.at[pl.ds(r0, ROWS_PER_TILE)])


def kernel(x, edge_index, W1, att_src1, att_dst1, b1, W2, att_src2, att_dst2,
           b2, W3, att_src3, att_dst3, b3, W4, att_src4, att_dst4, b4):
    n = x.shape[0]
    loop = jnp.arange(n, dtype=jnp.int32)
    pad = EP - E_TOT
    src = jnp.concatenate([edge_index[0], loop, jnp.zeros((pad,), jnp.int32)])
    dst = jnp.concatenate([edge_index[1], loop,
                           jnp.full((pad,), NP - 1, jnp.int32)])
    dst2 = dst.reshape(NW, E_LOC // EB, EB // 128, 128)
    x_pad = jnp.concatenate([x, jnp.zeros((NP - n, D), jnp.float32)], axis=0)
    z2d = jnp.zeros((64, D), jnp.float32)
    z1d = jnp.zeros((ROWS_PER_TILE,), jnp.float32)

    def edge(a_s, a_d, h):
        return _edge_kernel(src, dst, dst2, a_s, a_d, h, z2d, z1d)

    h, a_s, a_d = _first_dense(x_pad, W1, att_src1.reshape(1, D),
                               att_dst1.reshape(1, D))
    n0, n1, d0, d1 = edge(a_s, a_d, h)
    h, a_s, a_d = _mid_dense(n0, n1, d0, d1, b1.reshape(1, D), W2,
                             att_src2.reshape(1, D), att_dst2.reshape(1, D))
    n0, n1, d0, d1 = edge(a_s, a_d, h)
    h, a_s, a_d = _mid_dense(n0, n1, d0, d1, b2.reshape(1, D), W3,
                             att_src3.reshape(1, D), att_dst3.reshape(1, D))
    n0, n1, d0, d1 = edge(a_s, a_d, h)
    h, a_s, a_d = _mid_dense(n0, n1, d0, d1, b3.reshape(1, D), W4,
                             att_src4.reshape(1, D), att_dst4.reshape(1, D))
    n0, n1, d0, d1 = edge(a_s, a_d, h)
    return _final(n0, n1, d0, d1, b4.reshape(1, D))[:n]


# speedup vs baseline: 38.5689x; 1.3816x over previous
import functools

import jax
import jax.numpy as jnp
from jax import lax
from jax.experimental import pallas as pl
from jax.experimental.pallas import tpu as pltpu
from jax.experimental.pallas import tpu_sc as plsc

N_NODES = 10000
NP = 10240
D = 128
E_TOT = 320000 + N_NODES
NW = 32
E_LOC = 10368
EB = 1152
EP = NW * E_LOC
ROWS_PER_TILE = NP // 16
BM = 1024
DEN_EPS = 1e-16


def _proj(h, as_w, ad_w, h_ref, as_ref, ad_ref):
    h_ref[...] = h
    as_ref[...] = jnp.sum(h * as_w, axis=1)
    ad_ref[...] = jnp.sum(h * ad_w, axis=1)


def _first_dense_body(x_ref, w_ref, as_w_ref, ad_w_ref, h_ref, as_ref, ad_ref):
    h = jnp.dot(x_ref[...], w_ref[...], preferred_element_type=jnp.float32)
    _proj(h, as_w_ref[...], ad_w_ref[...], h_ref, as_ref, ad_ref)


def _mid_dense_body(n0_ref, n1_ref, d0_ref, d1_ref, b_ref, w_ref, as_w_ref,
                    ad_w_ref, h_ref, as_ref, ad_ref):
    den = d0_ref[...] + d1_ref[...] + DEN_EPS
    act = jnp.maximum((n0_ref[...] + n1_ref[...]) / den[:, None] + b_ref[...], 0.0)
    h = jnp.dot(act, w_ref[...], preferred_element_type=jnp.float32)
    _proj(h, as_w_ref[...], ad_w_ref[...], h_ref, as_ref, ad_ref)


def _final_body(n0_ref, n1_ref, d0_ref, d1_ref, b_ref, y_ref):
    den = d0_ref[...] + d1_ref[...] + DEN_EPS
    z = (n0_ref[...] + n1_ref[...]) / den[:, None] + b_ref[...]
    m = jnp.max(z, axis=1, keepdims=True)
    e = jnp.exp(z - m)
    y_ref[...] = e / jnp.sum(e, axis=1, keepdims=True)


_mat_spec = pl.BlockSpec((BM, D), lambda i: (i, 0))
_vec_spec = pl.BlockSpec((BM,), lambda i: (i,))
_row_spec = pl.BlockSpec((1, D), lambda i: (0, 0))
_w_spec = pl.BlockSpec((D, D), lambda i: (0, 0))

_dense_out = [
    jax.ShapeDtypeStruct((NP, D), jnp.float32),
    jax.ShapeDtypeStruct((NP,), jnp.float32),
    jax.ShapeDtypeStruct((NP,), jnp.float32),
]
_dense_out_spec = [_mat_spec, _vec_spec, _vec_spec]

_first_dense = pl.pallas_call(
    _first_dense_body,
    grid=(NP // BM,),
    in_specs=[_mat_spec, _w_spec, _row_spec, _row_spec],
    out_specs=_dense_out_spec,
    out_shape=_dense_out,
)

_mid_dense = pl.pallas_call(
    _mid_dense_body,
    grid=(NP // BM,),
    in_specs=[_mat_spec, _mat_spec, _vec_spec, _vec_spec, _row_spec, _w_spec,
              _row_spec, _row_spec],
    out_specs=_dense_out_spec,
    out_shape=_dense_out,
)

_final = pl.pallas_call(
    _final_body,
    grid=(NP // BM,),
    in_specs=[_mat_spec, _mat_spec, _vec_spec, _vec_spec, _row_spec],
    out_specs=_mat_spec,
    out_shape=jax.ShapeDtypeStruct((NP, D), jnp.float32),
)


_sc_mesh = plsc.VectorSubcoreMesh(core_axis_name="c", subcore_axis_name="s")


@functools.partial(
    pl.kernel,
    mesh=_sc_mesh,
    compiler_params=pltpu.CompilerParams(needs_layout_passes=False),
    out_type=[
        jax.ShapeDtypeStruct((NP, D), jnp.float32),
        jax.ShapeDtypeStruct((NP, D), jnp.float32),
        jax.ShapeDtypeStruct((NP,), jnp.float32),
        jax.ShapeDtypeStruct((NP,), jnp.float32),
    ],
    scratch_types=[
        pltpu.VMEM((EB,), jnp.int32),
        pltpu.VMEM((EB,), jnp.int32),
        pltpu.VMEM((EB // 128, 128), jnp.int32),
        pltpu.VMEM((NP,), jnp.float32),
        pltpu.VMEM((NP,), jnp.float32),
        pltpu.VMEM((EB,), jnp.float32),
        pltpu.VMEM((64, D), jnp.float32),
        pltpu.VMEM((64, D), jnp.float32),
        pltpu.VMEM((16, D), jnp.float32),
        pltpu.VMEM((16, D), jnp.float32),
        pltpu.VMEM_SHARED((NP, D), jnp.float32),
        pltpu.VMEM_SHARED((NP,), jnp.float32),
        pltpu.SemaphoreType.DMA,
        pltpu.SemaphoreType.DMA,
        pltpu.SemaphoreType.DMA,
    ],
)
def _edge_kernel(src_hbm, dst_hbm, dst2_hbm, as_hbm, ad_hbm, h_hbm, z2d_hbm,
                 z1d_hbm, num0_hbm, num1_hbm, den0_hbm, den1_hbm,
                 src_v, dst_v, dst_r, as_v, ad_v, p_v, rows_a, rows_b,
                 scaled_0, scaled_1, num_acc, den_acc, sem_a, sem_b, sem_s):
    c = lax.axis_index("c")
    s = lax.axis_index("s")
    w = s * 2 + c
    base = w * E_LOC
    r0 = s * ROWS_PER_TILE

    pltpu.sync_copy(as_hbm, as_v)
    pltpu.sync_copy(ad_hbm, ad_v)
    for k in range(ROWS_PER_TILE // 64):
        pltpu.sync_copy(z2d_hbm, num_acc.at[pl.ds(r0 + k * 64, 64)])
    pltpu.sync_copy(z1d_hbm, den_acc.at[pl.ds(r0, ROWS_PER_TILE)])
    plsc.subcore_barrier()

    scaled = (scaled_0, scaled_1)

    def issue(g, rows, sem):
        pltpu.async_copy(h_hbm.at[src_v.at[pl.ds(g * 64, 64)]], rows, sem)

    def process(g, rows, sem):
        pltpu.make_async_copy(h_hbm.at[pl.ds(0, 64)], rows, sem).wait()
        for q in range(4):
            i = g * 4 + q
            sv = src_v[pl.ds(i * 16, 16)]
            dv = dst_v[pl.ds(i * 16, 16)]
            a = plsc.load_gather(as_v, [sv])
            b = plsc.load_gather(ad_v, [dv])
            e = a + b
            e = jnp.where(e >= 0.0, e, 0.2 * e)
            p = jnp.exp(e)
            p_v[pl.ds(i * 16, 16)] = p
            sc = scaled[q % 2]
            if q >= 2:
                pltpu.make_async_copy(h_hbm.at[pl.ds(0, 16)], scaled_0,
                                      sem_s).wait()
            for k in range(16):
                pk = p[k]
                for j in range(D // 16):
                    sc[k, pl.ds(j * 16, 16)] = rows[q * 16 + k, pl.ds(j * 16, 16)] * pk
            pltpu.async_copy(sc, num_acc.at[dv], sem_s, add=True)
        for _ in range(2):
            pltpu.make_async_copy(h_hbm.at[pl.ds(0, 16)], scaled_0, sem_s).wait()

    def dchunk(j, carry):
        pltpu.sync_copy(p_v.at[pl.ds(j * 128, 128)], den_acc.at[dst_r.at[j]],
                        add=True)
        return carry

    ngather = EB // 64

    def block(blk, carry):
        off = base + blk * EB
        pltpu.sync_copy(src_hbm.at[pl.ds(off, EB)], src_v)
        pltpu.sync_copy(dst_hbm.at[pl.ds(off, EB)], dst_v)
        pltpu.sync_copy(dst2_hbm.at[w, blk], dst_r)
        issue(0, rows_a, sem_a)
        issue(1, rows_b, sem_b)

        def pair(t, c):
            process(2 * t, rows_a, sem_a)
            issue(2 * t + 2, rows_a, sem_a)
            process(2 * t + 1, rows_b, sem_b)
            issue(2 * t + 3, rows_b, sem_b)
            return c

        lax.fori_loop(0, ngather // 2 - 1, pair, 0)
        process(ngather - 2, rows_a, sem_a)
        process(ngather - 1, rows_b, sem_b)
        lax.fori_loop(0, EB // 128, dchunk, 0)
        return carry

    lax.fori_loop(0, E_LOC // EB, block, 0)
    plsc.subcore_barrier()

    @pl.when(c == 0)
    def _():
        pltpu.sync_copy(num_acc.at[pl.ds(r0, ROWS_PER_TILE)],
                        num0_hbm.at[pl.ds(r0, ROWS_PER_TILE)])
        pltpu.sync_copy(den_acc.at[pl.ds(r0, ROWS_PER_TILE)],
                        den0_hbm.at[pl.ds(r0, ROWS_PER_TILE)])

    @pl.when(c == 1)
    def _():
        pltpu.sync_copy(num_acc.at[pl.ds(r0, ROWS_PER_TILE)],
                        num1_hbm.at[pl.ds(r0, ROWS_PER_TILE)])
        pltpu.sync_copy(den_acc.at[pl.ds(r0, ROWS_PER_TILE)],
                        den1_hbm.at[pl.ds(r0, ROWS_PER_TILE)])


def kernel(x, edge_index, W1, att_src1, att_dst1, b1, W2, att_src2, att_dst2,
           b2, W3, att_src3, att_dst3, b3, W4, att_src4, att_dst4, b4):
    n = x.shape[0]
    loop = jnp.arange(n, dtype=jnp.int32)
    pad = EP - E_TOT
    src = jnp.concatenate([edge_index[0], loop, jnp.zeros((pad,), jnp.int32)])
    dst = jnp.concatenate([edge_index[1], loop,
                           jnp.full((pad,), NP - 1, jnp.int32)])
    dst2 = dst.reshape(NW, E_LOC // EB, EB // 128, 128)
    x_pad = jnp.concatenate([x, jnp.zeros((NP - n, D), jnp.float32)], axis=0)
    z2d = jnp.zeros((64, D), jnp.float32)
    z1d = jnp.zeros((ROWS_PER_TILE,), jnp.float32)

    def edge(a_s, a_d, h):
        return _edge_kernel(src, dst, dst2, a_s, a_d, h, z2d, z1d)

    h, a_s, a_d = _first_dense(x_pad, W1, att_src1.reshape(1, D),
                               att_dst1.reshape(1, D))
    n0, n1, d0, d1 = edge(a_s, a_d, h)
    h, a_s, a_d = _mid_dense(n0, n1, d0, d1, b1.reshape(1, D), W2,
                             att_src2.reshape(1, D), att_dst2.reshape(1, D))
    n0, n1, d0, d1 = edge(a_s, a_d, h)
    h, a_s, a_d = _mid_dense(n0, n1, d0, d1, b2.reshape(1, D), W3,
                             att_src3.reshape(1, D), att_dst3.reshape(1, D))
    n0, n1, d0, d1 = edge(a_s, a_d, h)
    h, a_s, a_d = _mid_dense(n0, n1, d0, d1, b3.reshape(1, D), W4,
                             att_src4.reshape(1, D), att_dst4.reshape(1, D))
    n0, n1, d0, d1 = edge(a_s, a_d, h)
    return _final(n0, n1, d0, d1, b4.reshape(1, D))[:n]


# speedup vs baseline: 39.0132x; 1.0115x over previous
import functools

import jax
import jax.numpy as jnp
from jax import lax
from jax.experimental import pallas as pl
from jax.experimental.pallas import tpu as pltpu
from jax.experimental.pallas import tpu_sc as plsc

N_NODES = 10000
NP = 10240
D = 128
E_TOT = 320000 + N_NODES
NW = 32
E_LOC = 10368
EB = 1152
EP = NW * E_LOC
ROWS_PER_TILE = NP // 16
BM = 1024
DEN_EPS = 1e-16


def _proj(h, as_w, ad_w, h_ref, as_ref, ad_ref):
    h_ref[...] = h
    as_ref[...] = jnp.sum(h * as_w, axis=1)
    ad_ref[...] = jnp.sum(h * ad_w, axis=1)


def _first_dense_body(x_ref, w_ref, as_w_ref, ad_w_ref, h_ref, as_ref, ad_ref):
    h = jnp.dot(x_ref[...], w_ref[...], preferred_element_type=jnp.float32)
    _proj(h, as_w_ref[...], ad_w_ref[...], h_ref, as_ref, ad_ref)


def _mid_dense_body(n0_ref, n1_ref, d0_ref, d1_ref, b_ref, w_ref, as_w_ref,
                    ad_w_ref, h_ref, as_ref, ad_ref):
    den = d0_ref[...] + d1_ref[...] + DEN_EPS
    act = jnp.maximum((n0_ref[...] + n1_ref[...]) / den[:, None] + b_ref[...], 0.0)
    h = jnp.dot(act, w_ref[...], preferred_element_type=jnp.float32)
    _proj(h, as_w_ref[...], ad_w_ref[...], h_ref, as_ref, ad_ref)


def _final_body(n0_ref, n1_ref, d0_ref, d1_ref, b_ref, y_ref):
    den = d0_ref[...] + d1_ref[...] + DEN_EPS
    z = (n0_ref[...] + n1_ref[...]) / den[:, None] + b_ref[...]
    m = jnp.max(z, axis=1, keepdims=True)
    e = jnp.exp(z - m)
    y_ref[...] = e / jnp.sum(e, axis=1, keepdims=True)


_mat_spec = pl.BlockSpec((BM, D), lambda i: (i, 0))
_vec_spec = pl.BlockSpec((BM,), lambda i: (i,))
_row_spec = pl.BlockSpec((1, D), lambda i: (0, 0))
_w_spec = pl.BlockSpec((D, D), lambda i: (0, 0))

_dense_out = [
    jax.ShapeDtypeStruct((NP, D), jnp.float32),
    jax.ShapeDtypeStruct((NP,), jnp.float32),
    jax.ShapeDtypeStruct((NP,), jnp.float32),
]
_dense_out_spec = [_mat_spec, _vec_spec, _vec_spec]

_first_dense = pl.pallas_call(
    _first_dense_body,
    grid=(NP // BM,),
    in_specs=[_mat_spec, _w_spec, _row_spec, _row_spec],
    out_specs=_dense_out_spec,
    out_shape=_dense_out,
)

_mid_dense = pl.pallas_call(
    _mid_dense_body,
    grid=(NP // BM,),
    in_specs=[_mat_spec, _mat_spec, _vec_spec, _vec_spec, _row_spec, _w_spec,
              _row_spec, _row_spec],
    out_specs=_dense_out_spec,
    out_shape=_dense_out,
)

_final = pl.pallas_call(
    _final_body,
    grid=(NP // BM,),
    in_specs=[_mat_spec, _mat_spec, _vec_spec, _vec_spec, _row_spec],
    out_specs=_mat_spec,
    out_shape=jax.ShapeDtypeStruct((NP, D), jnp.float32),
)


_sc_mesh = plsc.VectorSubcoreMesh(core_axis_name="c", subcore_axis_name="s")


@functools.partial(
    pl.kernel,
    mesh=_sc_mesh,
    compiler_params=pltpu.CompilerParams(needs_layout_passes=False),
    out_type=[
        jax.ShapeDtypeStruct((NP, D), jnp.float32),
        jax.ShapeDtypeStruct((NP, D), jnp.float32),
        jax.ShapeDtypeStruct((NP,), jnp.float32),
        jax.ShapeDtypeStruct((NP,), jnp.float32),
    ],
    scratch_types=[
        pltpu.VMEM((EB,), jnp.int32),
        pltpu.VMEM((EB,), jnp.int32),
        pltpu.VMEM((EB // 128, 128), jnp.int32),
        pltpu.VMEM((NP,), jnp.float32),
        pltpu.VMEM((NP,), jnp.float32),
        pltpu.VMEM((EB,), jnp.float32),
        pltpu.VMEM((64, D), jnp.float32),
        pltpu.VMEM((64, D), jnp.float32),
        pltpu.VMEM((16, D), jnp.float32),
        pltpu.VMEM((16, D), jnp.float32),
        pltpu.VMEM_SHARED((NP, D), jnp.float32),
        pltpu.VMEM_SHARED((NP,), jnp.float32),
        pltpu.SemaphoreType.DMA,
        pltpu.SemaphoreType.DMA,
        pltpu.SemaphoreType.DMA,
        pltpu.SemaphoreType.DMA,
    ],
)
def _edge_kernel(src_hbm, dst_hbm, dst2_hbm, as_hbm, ad_hbm, h_hbm, z2d_hbm,
                 z1d_hbm, num0_hbm, num1_hbm, den0_hbm, den1_hbm,
                 src_v, dst_v, dst_r, as_v, ad_v, p_v, rows_a, rows_b,
                 scaled_0, scaled_1, num_acc, den_acc, sem_a, sem_b, sem_s,
                 sem_d):
    c = lax.axis_index("c")
    s = lax.axis_index("s")
    w = s * 2 + c
    base = w * E_LOC
    r0 = s * ROWS_PER_TILE

    pltpu.sync_copy(as_hbm, as_v)
    pltpu.sync_copy(ad_hbm, ad_v)
    for k in range(ROWS_PER_TILE // 64):
        pltpu.sync_copy(z2d_hbm, num_acc.at[pl.ds(r0 + k * 64, 64)])
    pltpu.sync_copy(z1d_hbm, den_acc.at[pl.ds(r0, ROWS_PER_TILE)])
    plsc.subcore_barrier()

    scaled = (scaled_0, scaled_1)

    def issue(g, rows, sem):
        pltpu.async_copy(h_hbm.at[src_v.at[pl.ds(g * 64, 64)]], rows, sem)

    def process(g, rows, sem):
        pltpu.make_async_copy(h_hbm.at[pl.ds(0, 64)], rows, sem).wait()
        for q in range(4):
            i = g * 4 + q
            sv = src_v[pl.ds(i * 16, 16)]
            dv = dst_v[pl.ds(i * 16, 16)]
            a = plsc.load_gather(as_v, [sv])
            b = plsc.load_gather(ad_v, [dv])
            e = a + b
            e = jnp.where(e >= 0.0, e, 0.2 * e)
            p = jnp.exp(e)
            p_v[pl.ds(i * 16, 16)] = p
            sc = scaled[q % 2]
            if q >= 2:
                pltpu.make_async_copy(h_hbm.at[pl.ds(0, 16)], scaled_0,
                                      sem_s).wait()
            for k in range(16):
                pk = p[k]
                for j in range(D // 16):
                    sc[k, pl.ds(j * 16, 16)] = rows[q * 16 + k, pl.ds(j * 16, 16)] * pk
            pltpu.async_copy(sc, num_acc.at[dv], sem_s, add=True)
        for _ in range(2):
            pltpu.make_async_copy(h_hbm.at[pl.ds(0, 16)], scaled_0, sem_s).wait()

    def dchunk(j, carry):
        pltpu.async_copy(p_v.at[pl.ds(j * 128, 128)], den_acc.at[dst_r.at[j]],
                         sem_d, add=True)
        return carry

    ngather = EB // 64

    def block(blk, carry):
        off = base + blk * EB
        pltpu.sync_copy(src_hbm.at[pl.ds(off, EB)], src_v)
        pltpu.sync_copy(dst_hbm.at[pl.ds(off, EB)], dst_v)
        pltpu.sync_copy(dst2_hbm.at[w, blk], dst_r)
        issue(0, rows_a, sem_a)
        issue(1, rows_b, sem_b)

        def pair(t, c):
            process(2 * t, rows_a, sem_a)
            issue(2 * t + 2, rows_a, sem_a)
            process(2 * t + 1, rows_b, sem_b)
            issue(2 * t + 3, rows_b, sem_b)
            return c

        lax.fori_loop(0, ngather // 2 - 1, pair, 0)
        process(ngather - 2, rows_a, sem_a)
        process(ngather - 1, rows_b, sem_b)
        lax.fori_loop(0, EB // 128, dchunk, 0)
        def ddrain(j, carry):
            pltpu.make_async_copy(as_hbm.at[pl.ds(0, 128)],
                                  p_v.at[pl.ds(0, 128)], sem_d).wait()
            return carry
        lax.fori_loop(0, EB // 128, ddrain, 0)
        return carry

    lax.fori_loop(0, E_LOC // EB, block, 0)
    plsc.subcore_barrier()

    @pl.when(c == 0)
    def _():
        pltpu.sync_copy(num_acc.at[pl.ds(r0, ROWS_PER_TILE)],
                        num0_hbm.at[pl.ds(r0, ROWS_PER_TILE)])
        pltpu.sync_copy(den_acc.at[pl.ds(r0, ROWS_PER_TILE)],
                        den0_hbm.at[pl.ds(r0, ROWS_PER_TILE)])

    @pl.when(c == 1)
    def _():
        pltpu.sync_copy(num_acc.at[pl.ds(r0, ROWS_PER_TILE)],
                        num1_hbm.at[pl.ds(r0, ROWS_PER_TILE)])
        pltpu.sync_copy(den_acc.at[pl.ds(r0, ROWS_PER_TILE)],
                        den1_hbm.at[pl.ds(r0, ROWS_PER_TILE)])


def kernel(x, edge_index, W1, att_src1, att_dst1, b1, W2, att_src2, att_dst2,
           b2, W3, att_src3, att_dst3, b3, W4, att_src4, att_dst4, b4):
    n = x.shape[0]
    loop = jnp.arange(n, dtype=jnp.int32)
    pad = EP - E_TOT
    src = jnp.concatenate([edge_index[0], loop, jnp.zeros((pad,), jnp.int32)])
    dst = jnp.concatenate([edge_index[1], loop,
                           jnp.full((pad,), NP - 1, jnp.int32)])
    dst2 = dst.reshape(NW, E_LOC // EB, EB // 128, 128)
    x_pad = jnp.concatenate([x, jnp.zeros((NP - n, D), jnp.float32)], axis=0)
    z2d = jnp.zeros((64, D), jnp.float32)
    z1d = jnp.zeros((ROWS_PER_TILE,), jnp.float32)

    def edge(a_s, a_d, h):
        return _edge_kernel(src, dst, dst2, a_s, a_d, h, z2d, z1d)

    h, a_s, a_d = _first_dense(x_pad, W1, att_src1.reshape(1, D),
                               att_dst1.reshape(1, D))
    n0, n1, d0, d1 = edge(a_s, a_d, h)
    h, a_s, a_d = _mid_dense(n0, n1, d0, d1, b1.reshape(1, D), W2,
                             att_src2.reshape(1, D), att_dst2.reshape(1, D))
    n0, n1, d0, d1 = edge(a_s, a_d, h)
    h, a_s, a_d = _mid_dense(n0, n1, d0, d1, b2.reshape(1, D), W3,
                             att_src3.reshape(1, D), att_dst3.reshape(1, D))
    n0, n1, d0, d1 = edge(a_s, a_d, h)
    h, a_s, a_d = _mid_dense(n0, n1, d0, d1, b3.reshape(1, D), W4,
                             att_src4.reshape(1, D), att_dst4.reshape(1, D))
    n0, n1, d0, d1 = edge(a_s, a_d, h)
    return _final(n0, n1, d0, d1, b4.reshape(1, D))[:n]
